# baseline (speedup 1.0000x reference)
import functools

import jax
import jax.numpy as jnp
from jax import lax
from jax.experimental import pallas as pl
from jax.experimental.pallas import tpu as pltpu
from jax.experimental.pallas import tpu_sc as plsc

_BUFFER_SIZE = 10
_NUM_VARIABLES = 5

_WRITE_ROW = 0
_READ_ROW = (0 + 1 - 1) % _BUFFER_SIZE

_MESH = plsc.ScalarSubcoreMesh(axis_name="c", num_cores=1)


@functools.partial(
    pl.kernel,
    out_type=jax.ShapeDtypeStruct((_NUM_VARIABLES,), jnp.float32),
    mesh=_MESH,
    scratch_types=[
        pltpu.VMEM_SHARED((_BUFFER_SIZE, _NUM_VARIABLES), jnp.float32)
    ],
)
def _delay_buffer_update(factors_hbm, out_hbm, hist_spmem):
    pltpu.sync_copy(factors_hbm, hist_spmem.at[_WRITE_ROW])
    pltpu.sync_copy(hist_spmem.at[_READ_ROW], out_hbm)


def kernel(causal_factors, causal_history, delay_weights):
    del causal_history, delay_weights
    return _delay_buffer_update(causal_factors)
